# Initial kernel scaffold; baseline (speedup 1.0000x reference)
import jax
import jax.numpy as jnp
from jax.experimental import pallas as pl


def kernel(x, coords, edges, not_used, W_off1, b_off1, g_off1, be_off1, W_off2, b_off2, W_e1, b_e1, g_e1, be_e1, W_u1, b_u1, g_u1, be_u1, W_u2, b_u2):
    raise NotImplementedError("write your pallas kernel here")


# speedup vs baseline: 1.5952x; 1.5952x over previous
import functools

import jax
import jax.numpy as jnp
from jax import lax
from jax.experimental import pallas as pl
from jax.experimental.pallas import tpu as pltpu
from jax.experimental.pallas import tpu_sc as plsc

N = 10000
E = 320000
D = 128
EPS = 1e-3

NC = 2
NS = 16
NW = NC * NS
CPW = D // NW
CH = 3200
LANES = 16


def _mmul(a, b):
    return jnp.dot(a.astype(jnp.bfloat16), b.astype(jnp.bfloat16),
                   preferred_element_type=jnp.float32)


def _phase1_body(x_ref, coords_ref, Woff1_ref, boff1_ref, goff1_ref,
                 beoff1_ref, Woff2_ref, boff2_ref, A_ref, B_ref, be1_ref,
                 T1_ref, T2_ref):
    x = x_ref[...]
    coords = coords_ref[...]
    h = jnp.maximum(_mmul(x, Woff1_ref[...]) + boff1_ref[...], 0.0)
    mean = jnp.mean(h, axis=0, keepdims=True)
    var = jnp.mean((h - mean) ** 2, axis=0, keepdims=True)
    hb = goff1_ref[...] * (h - mean) * lax.rsqrt(var + EPS) + beoff1_ref[...]
    offset = _mmul(hb, Woff2_ref[...]) + boff2_ref[...]
    co = coords + offset
    T1_ref[...] = (_mmul(x, A_ref[...]) + _mmul(coords, B_ref[...])
                   + be1_ref[...])
    T2_ref[...] = _mmul(co, B_ref[...])


def _phase1(x, coords, W_off1, b_off1, g_off1, be_off1, W_off2, b_off2,
            A, B, b_e1):
    return pl.pallas_call(
        _phase1_body,
        out_shape=(jax.ShapeDtypeStruct((N, D), jnp.float32),
                   jax.ShapeDtypeStruct((N, D), jnp.float32)),
    )(x, coords, W_off1, b_off1, g_off1, be_off1, W_off2, b_off2, A, B, b_e1)


def _phase2_body(T1_hbm, T2_hbm, src_hbm, dst_hbm,
                 agg_hbm, stats_hbm, T1s, T2s, aggs, srcb, dstb, statsb):
    wid = lax.axis_index("s") * NC + lax.axis_index("c")
    pltpu.sync_copy(T1_hbm.at[wid], T1s)
    pltpu.sync_copy(T2_hbm.at[wid], T2s)
    neg = jnp.full((LANES,), -jnp.inf, jnp.float32)

    def init_body(i, _):
        aggs[pl.ds(pl.multiple_of(i * LANES, LANES), LANES)] = neg
        return 0
    lax.fori_loop(0, (N * CPW) // LANES, init_body, 0)

    def vec_body(i, carry):
        base = pl.multiple_of(i * LANES, LANES)
        src16 = srcb[pl.ds(base, LANES)] * CPW
        dst16 = dstb[pl.ds(base, LANES)] * CPW
        counts, _ = plsc.scan_count(dst16)
        sums = list(carry[:CPW])
        sqs = list(carry[CPW:])
        ys = []
        for c in range(CPW):
            t1 = plsc.load_gather(T1s, [src16 + c])
            t2 = plsc.load_gather(T2s, [dst16 + c])
            y = jnp.maximum(t1 - t2, 0.0)
            sums[c] = sums[c] + y
            sqs[c] = sqs[c] + y * y
            ys.append(y)

        def cond(j):
            return jnp.any(counts >= j)

        def pass_body(j):
            m = counts == j
            for c in range(CPW):
                old = plsc.load_gather(aggs, [dst16 + c])
                plsc.store_scatter(aggs, [dst16 + c],
                                   jnp.maximum(old, ys[c]), mask=m)
            return j + 1

        lax.while_loop(cond, pass_body, jnp.min(counts))
        return tuple(sums) + tuple(sqs)

    def chunk_body(ci, carry):
        off = pl.multiple_of(ci * CH, CH)
        pltpu.sync_copy(src_hbm.at[pl.ds(off, CH)], srcb)
        pltpu.sync_copy(dst_hbm.at[pl.ds(off, CH)], dstb)
        return lax.fori_loop(0, CH // LANES, vec_body, carry)

    zero = jnp.zeros((LANES,), jnp.float32)
    carry = lax.fori_loop(0, E // CH, chunk_body, (zero,) * (2 * CPW))
    for j in range(2 * CPW):
        statsb[pl.ds(j * LANES, LANES)] = carry[j]
    pltpu.sync_copy(statsb, stats_hbm.at[wid])
    pltpu.sync_copy(aggs, agg_hbm.at[wid])


def _phase2(T1t, T2t, src, dst):
    mesh = plsc.VectorSubcoreMesh(core_axis_name="c", subcore_axis_name="s")
    kfn = pl.kernel(
        _phase2_body,
        out_type=(jax.ShapeDtypeStruct((NW, N * CPW), jnp.float32),
                  jax.ShapeDtypeStruct((NW, 2 * CPW * LANES), jnp.float32)),
        mesh=mesh,
        compiler_params=pltpu.CompilerParams(needs_layout_passes=False),
        scratch_types=[
            pltpu.VMEM((N * CPW,), jnp.float32),
            pltpu.VMEM((N * CPW,), jnp.float32),
            pltpu.VMEM((N * CPW,), jnp.float32),
            pltpu.VMEM((CH,), jnp.int32),
            pltpu.VMEM((CH,), jnp.int32),
            pltpu.VMEM((2 * CPW * LANES,), jnp.float32),
        ],
    )
    return kfn(T1t, T2t, src, dst)


def _phase3_body(agg_ref, stats_ref, x_ref, ge1_ref, bee1_ref, Wu1_ref,
                 bu1_ref, gu1_ref, beu1_ref, Wu2_ref, bu2_ref, out_ref):
    stats = stats_ref[...]
    kk = lax.broadcasted_iota(jnp.int32, (D, NW), 0)
    ww = lax.broadcasted_iota(jnp.int32, (D, NW), 1)
    G1 = jnp.where(kk // CPW == ww, 1.0, 0.0)
    R = jnp.dot(G1, stats, preferred_element_type=jnp.float32, precision=lax.Precision.HIGHEST)
    km = lax.broadcasted_iota(jnp.int32, (D, D), 0)
    mm = lax.broadcasted_iota(jnp.int32, (D, D), 1)
    msk_s = mm // LANES == km % CPW
    msk_q = mm // LANES == CPW + km % CPW
    S_col = jnp.sum(jnp.where(msk_s, R, 0.0), axis=1, keepdims=True)
    Q_col = jnp.sum(jnp.where(msk_q, R, 0.0), axis=1, keepdims=True)
    I_d = jnp.where(km == mm, 1.0, 0.0)
    ones_row = jnp.ones((1, D), jnp.float32)
    S = jnp.dot(ones_row, S_col * I_d, preferred_element_type=jnp.float32, precision=lax.Precision.HIGHEST)
    Q = jnp.dot(ones_row, Q_col * I_d, preferred_element_type=jnp.float32, precision=lax.Precision.HIGHEST)
    em = S / E
    ev = jnp.maximum(Q / E - em * em, 0.0)
    agg = agg_ref[...]
    agg_bn = ge1_ref[...] * (agg - em) * lax.rsqrt(ev + EPS) + bee1_ref[...]
    u = jnp.maximum(_mmul(agg_bn, Wu1_ref[...]) + bu1_ref[...], 0.0)
    um = jnp.mean(u, axis=0, keepdims=True)
    uv = jnp.mean((u - um) ** 2, axis=0, keepdims=True)
    ub = gu1_ref[...] * (u - um) * lax.rsqrt(uv + EPS) + beu1_ref[...]
    out_ref[...] = _mmul(ub, Wu2_ref[...]) + bu2_ref[...] + x_ref[...]


def _phase3(agg, stats, x, g_e1, be_e1, W_u1, b_u1, g_u1, be_u1, W_u2, b_u2):
    return pl.pallas_call(
        _phase3_body,
        out_shape=jax.ShapeDtypeStruct((N, D), jnp.float32),
    )(agg, stats, x, g_e1, be_e1, W_u1, b_u1, g_u1, be_u1, W_u2, b_u2)


def kernel(x, coords, edges, not_used, W_off1, b_off1, g_off1, be_off1,
           W_off2, b_off2, W_e1, b_e1, g_e1, be_e1, W_u1, b_u1, g_u1, be_u1,
           W_u2, b_u2):
    src = edges[:, 0]
    dst = edges[:, 1]
    A = W_e1[:D]
    B = W_e1[D:]
    row = lambda v: v.reshape(1, -1)
    DEBUG_XLA_PHASE1 = False
    if DEBUG_XLA_PHASE1:
        h = jax.nn.relu(x @ W_off1 + b_off1)
        hm = jnp.mean(h, axis=0, keepdims=True)
        hv = jnp.mean((h - hm) ** 2, axis=0, keepdims=True)
        hb = g_off1 * (h - hm) * lax.rsqrt(hv + EPS) + be_off1
        co = coords + hb @ W_off2 + b_off2
        T1 = x @ A + coords @ B + b_e1
        T2 = co @ B
    else:
        T1, T2 = _phase1(x, coords, W_off1, row(b_off1), row(g_off1),
                         row(be_off1), W_off2, row(b_off2), A, B, row(b_e1))
    to_t = lambda T: (T.reshape(N, NW, CPW).transpose(1, 0, 2)
                      .reshape(NW, N * CPW))
    DEBUG_EMU_PHASE2 = False
    if DEBUG_EMU_PHASE2:
        y = jax.nn.relu(jnp.take(T1, src, axis=0) - jnp.take(T2, dst, axis=0))
        agg = jax.ops.segment_max(y, dst, num_segments=N)
        Sv = jnp.sum(y, axis=0).reshape(NW, CPW)
        Qv = jnp.sum(y * y, axis=0).reshape(NW, CPW)
        stats0 = jnp.zeros((NW, 2 * CPW, LANES), jnp.float32)
        stats0 = stats0.at[:, :CPW, 0].set(Sv).at[:, CPW:, 0].set(Qv)
        stats = stats0.reshape(NW, 2 * CPW * LANES)
    else:
        aggt, stats = _phase2(to_t(T1), to_t(T2), src, dst)
        agg = aggt.reshape(NW, N, CPW).transpose(1, 0, 2).reshape(N, D)
    DEBUG_XLA_PHASE3 = False
    if DEBUG_XLA_PHASE3:
        Sv2 = stats.reshape(NW, 2 * CPW, LANES)
        Sg = Sv2[:, :CPW, :].sum(-1).reshape(1, D)
        Qg = Sv2[:, CPW:, :].sum(-1).reshape(1, D)
        em = Sg / E
        ev = jnp.maximum(Qg / E - em * em, 0.0)
        agg_bn = g_e1 * (agg - em) * lax.rsqrt(ev + EPS) + be_e1
        u = jax.nn.relu(agg_bn @ W_u1 + b_u1)
        um = jnp.mean(u, axis=0, keepdims=True)
        uv = jnp.mean((u - um) ** 2, axis=0, keepdims=True)
        ub = g_u1 * (u - um) * lax.rsqrt(uv + EPS) + be_u1
        out = ub @ W_u2 + b_u2 + x
    else:
        out = _phase3(agg, stats, x, row(g_e1), row(be_e1), W_u1, row(b_u1),
                      row(g_u1), row(be_u1), W_u2, row(b_u2))
    return out


# speedup vs baseline: 3.4085x; 2.1367x over previous
import functools

import jax
import jax.numpy as jnp
from jax import lax
from jax.experimental import pallas as pl
from jax.experimental.pallas import tpu as pltpu
from jax.experimental.pallas import tpu_sc as plsc

N = 10000
E = 320000
D = 128
EPS = 1e-3

NC = 2
NS = 16
NW = NC * NS
CPW = D // NW
CH = 2000
LANES = 16


def _mmul(a, b):
    return jnp.dot(a.astype(jnp.bfloat16), b.astype(jnp.bfloat16),
                   preferred_element_type=jnp.float32)


def _phase1_body(x_ref, coords_ref, Woff1_ref, boff1_ref, goff1_ref,
                 beoff1_ref, Woff2_ref, boff2_ref, A_ref, B_ref, be1_ref,
                 T1_ref, T2_ref):
    x = x_ref[...]
    coords = coords_ref[...]
    h = jnp.maximum(_mmul(x, Woff1_ref[...]) + boff1_ref[...], 0.0)
    mean = jnp.mean(h, axis=0, keepdims=True)
    var = jnp.mean((h - mean) ** 2, axis=0, keepdims=True)
    hb = goff1_ref[...] * (h - mean) * lax.rsqrt(var + EPS) + beoff1_ref[...]
    offset = _mmul(hb, Woff2_ref[...]) + boff2_ref[...]
    co = coords + offset
    T1_ref[...] = (_mmul(x, A_ref[...]) + _mmul(coords, B_ref[...])
                   + be1_ref[...])
    T2_ref[...] = _mmul(co, B_ref[...])


def _phase1(x, coords, W_off1, b_off1, g_off1, be_off1, W_off2, b_off2,
            A, B, b_e1):
    return pl.pallas_call(
        _phase1_body,
        out_shape=(jax.ShapeDtypeStruct((N, D), jnp.float32),
                   jax.ShapeDtypeStruct((N, D), jnp.float32)),
    )(x, coords, W_off1, b_off1, g_off1, be_off1, W_off2, b_off2, A, B, b_e1)


NCHUNK = E // CH
NPAIR = NCHUNK // 2


def _phase2_body(T1_hbm, T2_hbm, src_hbm, dst_hbm,
                 agg_hbm, stats_hbm, T1s, T2s, aggs,
                 srcb0, dstb0, srcb1, dstb1, statsb, sem0, sem1):
    wid = lax.axis_index("s") * NC + lax.axis_index("c")
    pltpu.sync_copy(T1_hbm.at[wid], T1s)
    pltpu.sync_copy(T2_hbm.at[wid], T2s)
    neg = jnp.full((LANES,), -jnp.inf, jnp.float32)

    def init_body(i, _):
        aggs[pl.ds(pl.multiple_of(i * LANES, LANES), LANES)] = neg
        return 0
    lax.fori_loop(0, (N * CPW) // LANES, init_body, 0)

    probe, _ = plsc.scan_count(lax.iota(jnp.int32, LANES))
    j0 = jnp.min(probe)

    def issue(ci, sb, db, sem):
        off = ci * CH
        pltpu.async_copy(src_hbm.at[pl.ds(off, CH)], sb, sem)
        pltpu.async_copy(dst_hbm.at[pl.ds(off, CH)], db, sem)

    def wait(sb, db, sem):
        pltpu.make_async_copy(src_hbm.at[pl.ds(0, CH)], sb, sem).wait()
        pltpu.make_async_copy(dst_hbm.at[pl.ds(0, CH)], db, sem).wait()

    def vec_body_for(sbuf, dbuf):
        def vec_body(i, carry):
            base = pl.multiple_of(i * LANES, LANES)
            src16 = sbuf[pl.ds(base, LANES)]
            dst16 = dbuf[pl.ds(base, LANES)]
            counts, _ = plsc.scan_count(dst16)
            sums = list(carry[:CPW])
            sqs = list(carry[CPW:])
            ys = []
            for c in range(CPW):
                t1 = plsc.load_gather(T1s, [src16 + c])
                t2 = plsc.load_gather(T2s, [dst16 + c])
                y = jnp.maximum(t1 - t2, 0.0)
                sums[c] = sums[c] + y
                sqs[c] = sqs[c] + y * y
                ys.append(y)
            m1 = counts == j0
            for c in range(CPW):
                old = plsc.load_gather(aggs, [dst16 + c])
                plsc.store_scatter(aggs, [dst16 + c],
                                   jnp.maximum(old, ys[c]), mask=m1)

            @pl.when(jnp.any(jnp.logical_not(m1)))
            def _slow_path():
                def cond(j):
                    return jnp.any(counts >= j)

                def pass_body(j):
                    m = counts == j
                    for c in range(CPW):
                        old = plsc.load_gather(aggs, [dst16 + c])
                        plsc.store_scatter(aggs, [dst16 + c],
                                           jnp.maximum(old, ys[c]), mask=m)
                    return j + 1

                lax.while_loop(cond, pass_body, j0 + 1)

            return tuple(sums) + tuple(sqs)
        return vec_body

    vb0 = vec_body_for(srcb0, dstb0)
    vb1 = vec_body_for(srcb1, dstb1)

    issue(0, srcb0, dstb0, sem0)

    def pair_body(p, carry):
        issue(2 * p + 1, srcb1, dstb1, sem1)
        wait(srcb0, dstb0, sem0)
        carry = lax.fori_loop(0, CH // LANES, vb0, carry)

        @pl.when(p < NPAIR - 1)
        def _prefetch():
            issue(2 * p + 2, srcb0, dstb0, sem0)

        wait(srcb1, dstb1, sem1)
        carry = lax.fori_loop(0, CH // LANES, vb1, carry)
        return carry

    zero = jnp.zeros((LANES,), jnp.float32)
    carry = lax.fori_loop(0, NPAIR, pair_body, (zero,) * (2 * CPW))
    for j in range(2 * CPW):
        statsb[pl.ds(j * LANES, LANES)] = carry[j]
    pltpu.sync_copy(statsb, stats_hbm.at[wid])
    pltpu.sync_copy(aggs, agg_hbm.at[wid])


def _phase2(T1t, T2t, src, dst):
    mesh = plsc.VectorSubcoreMesh(core_axis_name="c", subcore_axis_name="s")
    kfn = pl.kernel(
        _phase2_body,
        out_type=(jax.ShapeDtypeStruct((NW, N * CPW), jnp.float32),
                  jax.ShapeDtypeStruct((NW, 2 * CPW * LANES), jnp.float32)),
        mesh=mesh,
        compiler_params=pltpu.CompilerParams(needs_layout_passes=False),
        scratch_types=[
            pltpu.VMEM((N * CPW,), jnp.float32),
            pltpu.VMEM((N * CPW,), jnp.float32),
            pltpu.VMEM((N * CPW,), jnp.float32),
            pltpu.VMEM((CH,), jnp.int32),
            pltpu.VMEM((CH,), jnp.int32),
            pltpu.VMEM((CH,), jnp.int32),
            pltpu.VMEM((CH,), jnp.int32),
            pltpu.VMEM((2 * CPW * LANES,), jnp.float32),
            pltpu.SemaphoreType.DMA,
            pltpu.SemaphoreType.DMA,
        ],
    )
    return kfn(T1t, T2t, src, dst)


def _phase3_body(agg_ref, stats_ref, x_ref, ge1_ref, bee1_ref, Wu1_ref,
                 bu1_ref, gu1_ref, beu1_ref, Wu2_ref, bu2_ref, out_ref):
    stats = stats_ref[...]
    kk = lax.broadcasted_iota(jnp.int32, (D, NW), 0)
    ww = lax.broadcasted_iota(jnp.int32, (D, NW), 1)
    G1 = jnp.where(kk // CPW == ww, 1.0, 0.0)
    R = jnp.dot(G1, stats, preferred_element_type=jnp.float32, precision=lax.Precision.HIGHEST)
    km = lax.broadcasted_iota(jnp.int32, (D, D), 0)
    mm = lax.broadcasted_iota(jnp.int32, (D, D), 1)
    msk_s = mm // LANES == km % CPW
    msk_q = mm // LANES == CPW + km % CPW
    S_col = jnp.sum(jnp.where(msk_s, R, 0.0), axis=1, keepdims=True)
    Q_col = jnp.sum(jnp.where(msk_q, R, 0.0), axis=1, keepdims=True)
    I_d = jnp.where(km == mm, 1.0, 0.0)
    ones_row = jnp.ones((1, D), jnp.float32)
    S = jnp.dot(ones_row, S_col * I_d, preferred_element_type=jnp.float32, precision=lax.Precision.HIGHEST)
    Q = jnp.dot(ones_row, Q_col * I_d, preferred_element_type=jnp.float32, precision=lax.Precision.HIGHEST)
    em = S / E
    ev = jnp.maximum(Q / E - em * em, 0.0)
    agg = agg_ref[...]
    agg_bn = ge1_ref[...] * (agg - em) * lax.rsqrt(ev + EPS) + bee1_ref[...]
    u = jnp.maximum(_mmul(agg_bn, Wu1_ref[...]) + bu1_ref[...], 0.0)
    um = jnp.mean(u, axis=0, keepdims=True)
    uv = jnp.mean((u - um) ** 2, axis=0, keepdims=True)
    ub = gu1_ref[...] * (u - um) * lax.rsqrt(uv + EPS) + beu1_ref[...]
    out_ref[...] = _mmul(ub, Wu2_ref[...]) + bu2_ref[...] + x_ref[...]


def _phase3(agg, stats, x, g_e1, be_e1, W_u1, b_u1, g_u1, be_u1, W_u2, b_u2):
    return pl.pallas_call(
        _phase3_body,
        out_shape=jax.ShapeDtypeStruct((N, D), jnp.float32),
    )(agg, stats, x, g_e1, be_e1, W_u1, b_u1, g_u1, be_u1, W_u2, b_u2)


def kernel(x, coords, edges, not_used, W_off1, b_off1, g_off1, be_off1,
           W_off2, b_off2, W_e1, b_e1, g_e1, be_e1, W_u1, b_u1, g_u1, be_u1,
           W_u2, b_u2):
    src = edges[:, 0] * CPW
    dst = edges[:, 1] * CPW
    A = W_e1[:D]
    B = W_e1[D:]
    row = lambda v: v.reshape(1, -1)
    T1, T2 = _phase1(x, coords, W_off1, row(b_off1), row(g_off1),
                     row(be_off1), W_off2, row(b_off2), A, B, row(b_e1))
    to_t = lambda T: (T.reshape(N, NW, CPW).transpose(1, 0, 2)
                      .reshape(NW, N * CPW))
    aggt, stats = _phase2(to_t(T1), to_t(T2), src, dst)
    agg = aggt.reshape(NW, N, CPW).transpose(1, 0, 2).reshape(N, D)
    out = _phase3(agg, stats, x, row(g_e1), row(be_e1), W_u1, row(b_u1),
                  row(g_u1), row(be_u1), W_u2, row(b_u2))
    return out


# speedup vs baseline: 3.7335x; 1.0953x over previous
import functools

import jax
import jax.numpy as jnp
from jax import lax
from jax.experimental import pallas as pl
from jax.experimental.pallas import tpu as pltpu
from jax.experimental.pallas import tpu_sc as plsc

N = 10000
E = 320000
D = 128
EPS = 1e-3

NC = 2
NS = 16
NW = NC * NS
CPW = D // NW
CH = 2000
LANES = 16


def _mmul(a, b):
    return jnp.dot(a.astype(jnp.bfloat16), b.astype(jnp.bfloat16),
                   preferred_element_type=jnp.float32)


def _phase1_body(x_ref, coords_ref, Woff1_ref, boff1_ref, goff1_ref,
                 beoff1_ref, Woff2_ref, boff2_ref, A_ref, B_ref, be1_ref,
                 T1_ref, T2_ref):
    x = x_ref[...]
    coords = coords_ref[...]
    h = jnp.maximum(_mmul(x, Woff1_ref[...]) + boff1_ref[...], 0.0)
    mean = jnp.mean(h, axis=0, keepdims=True)
    var = jnp.mean((h - mean) ** 2, axis=0, keepdims=True)
    hb = goff1_ref[...] * (h - mean) * lax.rsqrt(var + EPS) + beoff1_ref[...]
    offset = _mmul(hb, Woff2_ref[...]) + boff2_ref[...]
    co = coords + offset
    T1_ref[...] = (_mmul(x, A_ref[...]) + _mmul(coords, B_ref[...])
                   + be1_ref[...])
    T2_ref[...] = _mmul(co, B_ref[...])


def _phase1(x, coords, W_off1, b_off1, g_off1, be_off1, W_off2, b_off2,
            A, B, b_e1):
    return pl.pallas_call(
        _phase1_body,
        out_shape=(jax.ShapeDtypeStruct((N, D), jnp.float32),
                   jax.ShapeDtypeStruct((N, D), jnp.float32)),
    )(x, coords, W_off1, b_off1, g_off1, be_off1, W_off2, b_off2, A, B, b_e1)


NCHUNK = E // CH
NPAIR = NCHUNK // 2


def _phase2_body(T1_hbm, T2_hbm, src_hbm, dst_hbm,
                 agg_hbm, stats_hbm, T1s, T2s, aggs,
                 srcb0, dstb0, srcb1, dstb1, statsb, sem0, sem1):
    wid = lax.axis_index("s") * NC + lax.axis_index("c")
    pltpu.sync_copy(T1_hbm.at[wid], T1s)
    pltpu.sync_copy(T2_hbm.at[wid], T2s)
    neg = jnp.full((LANES,), -jnp.inf, jnp.float32)

    def init_body(i, _):
        aggs[pl.ds(pl.multiple_of(i * LANES, LANES), LANES)] = neg
        return 0
    lax.fori_loop(0, (N * CPW) // LANES, init_body, 0)

    probe, _ = plsc.scan_count(lax.iota(jnp.int32, LANES))
    j0 = jnp.min(probe)

    def issue(ci, sb, db, sem):
        off = ci * CH
        pltpu.async_copy(src_hbm.at[pl.ds(off, CH)], sb, sem)
        pltpu.async_copy(dst_hbm.at[pl.ds(off, CH)], db, sem)

    def wait(sb, db, sem):
        pltpu.make_async_copy(src_hbm.at[pl.ds(0, CH)], sb, sem).wait()
        pltpu.make_async_copy(dst_hbm.at[pl.ds(0, CH)], db, sem).wait()

    def vec_body_for(sbuf, dbuf):
        def vec_body(i, carry):
            base = pl.multiple_of(i * LANES, LANES)
            src16 = sbuf[pl.ds(base, LANES)]
            dst16 = dbuf[pl.ds(base, LANES)]
            counts, _ = plsc.scan_count(dst16)
            sums = list(carry[:CPW])
            sqs = list(carry[CPW:])
            ys = []
            for c in range(CPW):
                t1 = plsc.load_gather(T1s, [src16 + c])
                t2 = plsc.load_gather(T2s, [dst16 + c])
                y = jnp.maximum(t1 - t2, 0.0)
                sums[c] = sums[c] + y
                sqs[c] = sqs[c] + y * y
                ys.append(y)
            m1 = counts == j0
            olds = [plsc.load_gather(aggs, [dst16 + c]) for c in range(CPW)]
            for c in range(CPW):
                plsc.store_scatter(aggs, [dst16 + c],
                                   jnp.maximum(olds[c], ys[c]), mask=m1)

            @pl.when(jnp.any(jnp.logical_not(m1)))
            def _slow_path():
                def cond(j):
                    return jnp.any(counts >= j)

                def pass_body(j):
                    m = counts == j
                    for c in range(CPW):
                        old = plsc.load_gather(aggs, [dst16 + c])
                        plsc.store_scatter(aggs, [dst16 + c],
                                           jnp.maximum(old, ys[c]), mask=m)
                    return j + 1

                lax.while_loop(cond, pass_body, j0 + 1)

            return tuple(sums) + tuple(sqs)
        return vec_body

    vb0 = vec_body_for(srcb0, dstb0)
    vb1 = vec_body_for(srcb1, dstb1)

    issue(0, srcb0, dstb0, sem0)

    def pair_body(p, carry):
        issue(2 * p + 1, srcb1, dstb1, sem1)
        wait(srcb0, dstb0, sem0)
        carry = lax.fori_loop(0, CH // LANES, vb0, carry, unroll=2)

        @pl.when(p < NPAIR - 1)
        def _prefetch():
            issue(2 * p + 2, srcb0, dstb0, sem0)

        wait(srcb1, dstb1, sem1)
        carry = lax.fori_loop(0, CH // LANES, vb1, carry, unroll=2)
        return carry

    zero = jnp.zeros((LANES,), jnp.float32)
    carry = lax.fori_loop(0, NPAIR, pair_body, (zero,) * (2 * CPW))
    for j in range(2 * CPW):
        statsb[pl.ds(j * LANES, LANES)] = carry[j]
    pltpu.sync_copy(statsb, stats_hbm.at[wid])
    pltpu.sync_copy(aggs, agg_hbm.at[wid])


def _phase2(T1t, T2t, src, dst):
    mesh = plsc.VectorSubcoreMesh(core_axis_name="c", subcore_axis_name="s")
    kfn = pl.kernel(
        _phase2_body,
        out_type=(jax.ShapeDtypeStruct((NW, N * CPW), jnp.float32),
                  jax.ShapeDtypeStruct((NW, 2 * CPW * LANES), jnp.float32)),
        mesh=mesh,
        compiler_params=pltpu.CompilerParams(needs_layout_passes=False),
        scratch_types=[
            pltpu.VMEM((N * CPW,), jnp.float32),
            pltpu.VMEM((N * CPW,), jnp.float32),
            pltpu.VMEM((N * CPW,), jnp.float32),
            pltpu.VMEM((CH,), jnp.int32),
            pltpu.VMEM((CH,), jnp.int32),
            pltpu.VMEM((CH,), jnp.int32),
            pltpu.VMEM((CH,), jnp.int32),
            pltpu.VMEM((2 * CPW * LANES,), jnp.float32),
            pltpu.SemaphoreType.DMA,
            pltpu.SemaphoreType.DMA,
        ],
    )
    return kfn(T1t, T2t, src, dst)


def _phase3_body(agg_ref, stats_ref, x_ref, ge1_ref, bee1_ref, Wu1_ref,
                 bu1_ref, gu1_ref, beu1_ref, Wu2_ref, bu2_ref, out_ref):
    stats = stats_ref[...]
    kk = lax.broadcasted_iota(jnp.int32, (D, NW), 0)
    ww = lax.broadcasted_iota(jnp.int32, (D, NW), 1)
    G1 = jnp.where(kk // CPW == ww, 1.0, 0.0)
    R = jnp.dot(G1, stats, preferred_element_type=jnp.float32, precision=lax.Precision.HIGHEST)
    km = lax.broadcasted_iota(jnp.int32, (D, D), 0)
    mm = lax.broadcasted_iota(jnp.int32, (D, D), 1)
    msk_s = mm // LANES == km % CPW
    msk_q = mm // LANES == CPW + km % CPW
    S_col = jnp.sum(jnp.where(msk_s, R, 0.0), axis=1, keepdims=True)
    Q_col = jnp.sum(jnp.where(msk_q, R, 0.0), axis=1, keepdims=True)
    I_d = jnp.where(km == mm, 1.0, 0.0)
    ones_row = jnp.ones((1, D), jnp.float32)
    S = jnp.dot(ones_row, S_col * I_d, preferred_element_type=jnp.float32, precision=lax.Precision.HIGHEST)
    Q = jnp.dot(ones_row, Q_col * I_d, preferred_element_type=jnp.float32, precision=lax.Precision.HIGHEST)
    em = S / E
    ev = jnp.maximum(Q / E - em * em, 0.0)
    agg = agg_ref[...]
    agg_bn = ge1_ref[...] * (agg - em) * lax.rsqrt(ev + EPS) + bee1_ref[...]
    u = jnp.maximum(_mmul(agg_bn, Wu1_ref[...]) + bu1_ref[...], 0.0)
    um = jnp.mean(u, axis=0, keepdims=True)
    uv = jnp.mean((u - um) ** 2, axis=0, keepdims=True)
    ub = gu1_ref[...] * (u - um) * lax.rsqrt(uv + EPS) + beu1_ref[...]
    out_ref[...] = _mmul(ub, Wu2_ref[...]) + bu2_ref[...] + x_ref[...]


def _phase3(agg, stats, x, g_e1, be_e1, W_u1, b_u1, g_u1, be_u1, W_u2, b_u2):
    return pl.pallas_call(
        _phase3_body,
        out_shape=jax.ShapeDtypeStruct((N, D), jnp.float32),
    )(agg, stats, x, g_e1, be_e1, W_u1, b_u1, g_u1, be_u1, W_u2, b_u2)


def kernel(x, coords, edges, not_used, W_off1, b_off1, g_off1, be_off1,
           W_off2, b_off2, W_e1, b_e1, g_e1, be_e1, W_u1, b_u1, g_u1, be_u1,
           W_u2, b_u2):
    src = edges[:, 0] * CPW
    dst = edges[:, 1] * CPW
    A = W_e1[:D]
    B = W_e1[D:]
    row = lambda v: v.reshape(1, -1)
    T1, T2 = _phase1(x, coords, W_off1, row(b_off1), row(g_off1),
                     row(be_off1), W_off2, row(b_off2), A, B, row(b_e1))
    to_t = lambda T: (T.reshape(N, NW, CPW).transpose(1, 0, 2)
                      .reshape(NW, N * CPW))
    aggt, stats = _phase2(to_t(T1), to_t(T2), src, dst)
    agg = aggt.reshape(NW, N, CPW).transpose(1, 0, 2).reshape(N, D)
    out = _phase3(agg, stats, x, row(g_e1), row(be_e1), W_u1, row(b_u1),
                  row(g_u1), row(be_u1), W_u2, row(b_u2))
    return out


# speedup vs baseline: 3.8566x; 1.0330x over previous
import functools

import jax
import jax.numpy as jnp
from jax import lax
from jax.experimental import pallas as pl
from jax.experimental.pallas import tpu as pltpu
from jax.experimental.pallas import tpu_sc as plsc

N = 10000
E = 320000
D = 128
EPS = 1e-3

NC = 2
NS = 16
NW = NC * NS
CPW = D // NW
CH = 2000
LANES = 16


def _mmul(a, b):
    return jnp.dot(a.astype(jnp.bfloat16), b.astype(jnp.bfloat16),
                   preferred_element_type=jnp.float32)


def _phase1_body(x_ref, coords_ref, Woff1_ref, boff1_ref, goff1_ref,
                 beoff1_ref, Woff2_ref, boff2_ref, A_ref, B_ref, be1_ref,
                 T1_ref, T2_ref):
    x = x_ref[...]
    coords = coords_ref[...]
    h = jnp.maximum(_mmul(x, Woff1_ref[...]) + boff1_ref[...], 0.0)
    mean = jnp.mean(h, axis=0, keepdims=True)
    var = jnp.mean((h - mean) ** 2, axis=0, keepdims=True)
    hb = goff1_ref[...] * (h - mean) * lax.rsqrt(var + EPS) + beoff1_ref[...]
    offset = _mmul(hb, Woff2_ref[...]) + boff2_ref[...]
    co = coords + offset
    T1_ref[...] = (_mmul(x, A_ref[...]) + _mmul(coords, B_ref[...])
                   + be1_ref[...])
    T2_ref[...] = _mmul(co, B_ref[...])


def _phase1(x, coords, W_off1, b_off1, g_off1, be_off1, W_off2, b_off2,
            A, B, b_e1):
    return pl.pallas_call(
        _phase1_body,
        out_shape=(jax.ShapeDtypeStruct((N, D), jnp.float32),
                   jax.ShapeDtypeStruct((N, D), jnp.float32)),
    )(x, coords, W_off1, b_off1, g_off1, be_off1, W_off2, b_off2, A, B, b_e1)


NCHUNK = E // CH
NPAIR = NCHUNK // 2


def _phase2_body(T1_hbm, T2_hbm, src_hbm, dst_hbm,
                 agg_hbm, stats_hbm, T1s, T2s, aggs,
                 srcb0, dstb0, srcb1, dstb1, statsb, sem0, sem1):
    wid = lax.axis_index("s") * NC + lax.axis_index("c")
    pltpu.sync_copy(T1_hbm.at[wid], T1s)
    pltpu.sync_copy(T2_hbm.at[wid], T2s)
    neg = jnp.full((LANES,), -jnp.inf, jnp.float32)

    def init_body(i, _):
        aggs[pl.ds(pl.multiple_of(i * LANES, LANES), LANES)] = neg
        return 0
    lax.fori_loop(0, (N * CPW) // LANES, init_body, 0)

    probe, _ = plsc.scan_count(lax.iota(jnp.int32, LANES))
    j0 = jnp.min(probe)

    def issue(ci, sb, db, sem):
        off = ci * CH
        pltpu.async_copy(src_hbm.at[pl.ds(off, CH)], sb, sem)
        pltpu.async_copy(dst_hbm.at[pl.ds(off, CH)], db, sem)

    def wait(sb, db, sem):
        pltpu.make_async_copy(src_hbm.at[pl.ds(0, CH)], sb, sem).wait()
        pltpu.make_async_copy(dst_hbm.at[pl.ds(0, CH)], db, sem).wait()

    NVEC = CH // LANES

    def prefetch(sbuf, dbuf, i):
        base = pl.multiple_of(i * LANES, LANES)
        s16 = sbuf[pl.ds(base, LANES)]
        d16 = dbuf[pl.ds(base, LANES)]
        cnt, _ = plsc.scan_count(d16)
        dup = jnp.any(cnt != j0)
        return s16, d16, cnt, dup

    def vec_body_for(sbuf, dbuf):
        def vec_body(i, carry):
            sums = list(carry[:CPW])
            sqs = list(carry[CPW:2 * CPW])
            src16, dst16, counts, dup = carry[2 * CPW:]
            nxt = prefetch(sbuf, dbuf, jnp.minimum(i + 1, NVEC - 1))
            ys = []
            for c in range(CPW):
                t1 = plsc.load_gather(T1s, [src16 + c])
                t2 = plsc.load_gather(T2s, [dst16 + c])
                y = jnp.maximum(t1 - t2, 0.0)
                sums[c] = sums[c] + y
                sqs[c] = sqs[c] + y * y
                ys.append(y)
            m1 = counts == j0
            olds = [plsc.load_gather(aggs, [dst16 + c]) for c in range(CPW)]
            for c in range(CPW):
                plsc.store_scatter(aggs, [dst16 + c],
                                   jnp.maximum(olds[c], ys[c]), mask=m1)

            @pl.when(dup)
            def _slow_path():
                def cond(j):
                    return jnp.any(counts >= j)

                def pass_body(j):
                    m = counts == j
                    for c in range(CPW):
                        old = plsc.load_gather(aggs, [dst16 + c])
                        plsc.store_scatter(aggs, [dst16 + c],
                                           jnp.maximum(old, ys[c]), mask=m)
                    return j + 1

                lax.while_loop(cond, pass_body, j0 + 1)

            return tuple(sums) + tuple(sqs) + nxt
        return vec_body

    vb0 = vec_body_for(srcb0, dstb0)
    vb1 = vec_body_for(srcb1, dstb1)

    issue(0, srcb0, dstb0, sem0)

    def pair_body(p, carry):
        issue(2 * p + 1, srcb1, dstb1, sem1)
        wait(srcb0, dstb0, sem0)
        carry = carry[:2 * CPW] + prefetch(srcb0, dstb0, 0)
        carry = lax.fori_loop(0, NVEC, vb0, carry, unroll=2)

        @pl.when(p < NPAIR - 1)
        def _prefetch_chunk():
            issue(2 * p + 2, srcb0, dstb0, sem0)

        wait(srcb1, dstb1, sem1)
        carry = carry[:2 * CPW] + prefetch(srcb1, dstb1, 0)
        carry = lax.fori_loop(0, NVEC, vb1, carry, unroll=2)
        return carry

    zero = jnp.zeros((LANES,), jnp.float32)
    zi = jnp.zeros((LANES,), jnp.int32)
    carry0 = (zero,) * (2 * CPW) + (zi, zi, zi, jnp.bool_(False))
    carry = lax.fori_loop(0, NPAIR, pair_body, carry0)
    for j in range(2 * CPW):
        statsb[pl.ds(j * LANES, LANES)] = carry[j]
    pltpu.sync_copy(statsb, stats_hbm.at[wid])
    pltpu.sync_copy(aggs, agg_hbm.at[wid])


def _phase2(T1t, T2t, src, dst):
    mesh = plsc.VectorSubcoreMesh(core_axis_name="c", subcore_axis_name="s")
    kfn = pl.kernel(
        _phase2_body,
        out_type=(jax.ShapeDtypeStruct((NW, N * CPW), jnp.float32),
                  jax.ShapeDtypeStruct((NW, 2 * CPW * LANES), jnp.float32)),
        mesh=mesh,
        compiler_params=pltpu.CompilerParams(needs_layout_passes=False),
        scratch_types=[
            pltpu.VMEM((N * CPW,), jnp.float32),
            pltpu.VMEM((N * CPW,), jnp.float32),
            pltpu.VMEM((N * CPW,), jnp.float32),
            pltpu.VMEM((CH,), jnp.int32),
            pltpu.VMEM((CH,), jnp.int32),
            pltpu.VMEM((CH,), jnp.int32),
            pltpu.VMEM((CH,), jnp.int32),
            pltpu.VMEM((2 * CPW * LANES,), jnp.float32),
            pltpu.SemaphoreType.DMA,
            pltpu.SemaphoreType.DMA,
        ],
    )
    return kfn(T1t, T2t, src, dst)


def _phase3_body(agg_ref, stats_ref, x_ref, ge1_ref, bee1_ref, Wu1_ref,
                 bu1_ref, gu1_ref, beu1_ref, Wu2_ref, bu2_ref, out_ref):
    stats = stats_ref[...]
    kk = lax.broadcasted_iota(jnp.int32, (D, NW), 0)
    ww = lax.broadcasted_iota(jnp.int32, (D, NW), 1)
    G1 = jnp.where(kk // CPW == ww, 1.0, 0.0)
    R = jnp.dot(G1, stats, preferred_element_type=jnp.float32, precision=lax.Precision.HIGHEST)
    km = lax.broadcasted_iota(jnp.int32, (D, D), 0)
    mm = lax.broadcasted_iota(jnp.int32, (D, D), 1)
    msk_s = mm // LANES == km % CPW
    msk_q = mm // LANES == CPW + km % CPW
    S_col = jnp.sum(jnp.where(msk_s, R, 0.0), axis=1, keepdims=True)
    Q_col = jnp.sum(jnp.where(msk_q, R, 0.0), axis=1, keepdims=True)
    I_d = jnp.where(km == mm, 1.0, 0.0)
    ones_row = jnp.ones((1, D), jnp.float32)
    S = jnp.dot(ones_row, S_col * I_d, preferred_element_type=jnp.float32, precision=lax.Precision.HIGHEST)
    Q = jnp.dot(ones_row, Q_col * I_d, preferred_element_type=jnp.float32, precision=lax.Precision.HIGHEST)
    em = S / E
    ev = jnp.maximum(Q / E - em * em, 0.0)
    agg = agg_ref[...]
    agg_bn = ge1_ref[...] * (agg - em) * lax.rsqrt(ev + EPS) + bee1_ref[...]
    u = jnp.maximum(_mmul(agg_bn, Wu1_ref[...]) + bu1_ref[...], 0.0)
    um = jnp.mean(u, axis=0, keepdims=True)
    uv = jnp.mean((u - um) ** 2, axis=0, keepdims=True)
    ub = gu1_ref[...] * (u - um) * lax.rsqrt(uv + EPS) + beu1_ref[...]
    out_ref[...] = _mmul(ub, Wu2_ref[...]) + bu2_ref[...] + x_ref[...]


def _phase3(agg, stats, x, g_e1, be_e1, W_u1, b_u1, g_u1, be_u1, W_u2, b_u2):
    return pl.pallas_call(
        _phase3_body,
        out_shape=jax.ShapeDtypeStruct((N, D), jnp.float32),
    )(agg, stats, x, g_e1, be_e1, W_u1, b_u1, g_u1, be_u1, W_u2, b_u2)


def kernel(x, coords, edges, not_used, W_off1, b_off1, g_off1, be_off1,
           W_off2, b_off2, W_e1, b_e1, g_e1, be_e1, W_u1, b_u1, g_u1, be_u1,
           W_u2, b_u2):
    src = edges[:, 0] * CPW
    dst = edges[:, 1] * CPW
    A = W_e1[:D]
    B = W_e1[D:]
    row = lambda v: v.reshape(1, -1)
    T1, T2 = _phase1(x, coords, W_off1, row(b_off1), row(g_off1),
                     row(be_off1), W_off2, row(b_off2), A, B, row(b_e1))
    to_t = lambda T: (T.reshape(N, NW, CPW).transpose(1, 0, 2)
                      .reshape(NW, N * CPW))
    aggt, stats = _phase2(to_t(T1), to_t(T2), src, dst)
    agg = aggt.reshape(NW, N, CPW).transpose(1, 0, 2).reshape(N, D)
    out = _phase3(agg, stats, x, row(g_e1), row(be_e1), W_u1, row(b_u1),
                  row(g_u1), row(be_u1), W_u2, row(b_u2))
    return out


# speedup vs baseline: 4.0400x; 1.0476x over previous
import functools

import jax
import jax.numpy as jnp
from jax import lax
from jax.experimental import pallas as pl
from jax.experimental.pallas import tpu as pltpu
from jax.experimental.pallas import tpu_sc as plsc

N = 10000
E = 320000
D = 128
EPS = 1e-3

NC = 2
NS = 16
NW = NC * NS
CPW = D // NW
CH = 2000
LANES = 16


def _mmul(a, b):
    return jnp.dot(a.astype(jnp.bfloat16), b.astype(jnp.bfloat16),
                   preferred_element_type=jnp.float32)


def _phase1_body(x_ref, coords_ref, Woff1_ref, boff1_ref, goff1_ref,
                 beoff1_ref, Woff2_ref, boff2_ref, A_ref, B_ref, be1_ref,
                 T1_ref, T2_ref):
    x = x_ref[...]
    coords = coords_ref[...]
    h = jnp.maximum(_mmul(x, Woff1_ref[...]) + boff1_ref[...], 0.0)
    mean = jnp.mean(h, axis=0, keepdims=True)
    var = jnp.mean((h - mean) ** 2, axis=0, keepdims=True)
    hb = goff1_ref[...] * (h - mean) * lax.rsqrt(var + EPS) + beoff1_ref[...]
    offset = _mmul(hb, Woff2_ref[...]) + boff2_ref[...]
    co = coords + offset
    T1_ref[...] = (_mmul(x, A_ref[...]) + _mmul(coords, B_ref[...])
                   + be1_ref[...])
    T2_ref[...] = _mmul(co, B_ref[...])


def _phase1(x, coords, W_off1, b_off1, g_off1, be_off1, W_off2, b_off2,
            A, B, b_e1):
    return pl.pallas_call(
        _phase1_body,
        out_shape=(jax.ShapeDtypeStruct((N, D), jnp.float32),
                   jax.ShapeDtypeStruct((N, D), jnp.float32)),
    )(x, coords, W_off1, b_off1, g_off1, be_off1, W_off2, b_off2, A, B, b_e1)


NCHUNK = E // CH
NPAIR = NCHUNK // 2


def _phase2_body(T1_hbm, T2_hbm, src_hbm, dst_hbm,
                 agg_hbm, stats_hbm, T1s, T2s, aggs,
                 srcb0, dstb0, srcb1, dstb1, statsb, sem0, sem1):
    wid = lax.axis_index("s") * NC + lax.axis_index("c")
    pltpu.sync_copy(T1_hbm.at[wid], T1s)
    pltpu.sync_copy(T2_hbm.at[wid], T2s)
    neg = jnp.full((LANES,), -jnp.inf, jnp.float32)

    def init_body(i, _):
        aggs[pl.ds(pl.multiple_of(i * LANES, LANES), LANES)] = neg
        return 0
    lax.fori_loop(0, (N * CPW) // LANES, init_body, 0)

    probe, _ = plsc.scan_count(lax.iota(jnp.int32, LANES))
    j0 = jnp.min(probe)

    def issue(ci, sb, db, sem):
        off = ci * CH
        pltpu.async_copy(src_hbm.at[pl.ds(off, CH)], sb, sem)
        pltpu.async_copy(dst_hbm.at[pl.ds(off, CH)], db, sem)

    def wait(sb, db, sem):
        pltpu.make_async_copy(src_hbm.at[pl.ds(0, CH)], sb, sem).wait()
        pltpu.make_async_copy(dst_hbm.at[pl.ds(0, CH)], db, sem).wait()

    NVEC = CH // LANES

    def load_idx(sbuf, dbuf, i):
        base = pl.multiple_of(i * LANES, LANES)
        return sbuf[pl.ds(base, LANES)], dbuf[pl.ds(base, LANES)]

    def count_of(d16):
        cnt, _ = plsc.scan_count(d16)
        dup = jnp.any(cnt != j0)
        return cnt, dup

    def vec_body_for(sbuf, dbuf):
        def vec_body(i, carry):
            sums = list(carry[:CPW])
            sqs = list(carry[CPW:2 * CPW])
            src16, dst16, counts, dup, s_n, d_n = carry[2 * CPW:]
            cd_n = count_of(d_n)
            sd_nn = load_idx(sbuf, dbuf, jnp.minimum(i + 2, NVEC - 1))
            nxt = (s_n, d_n) + cd_n + sd_nn
            ys = []
            for c in range(CPW):
                t1 = plsc.load_gather(T1s, [src16 + c])
                t2 = plsc.load_gather(T2s, [dst16 + c])
                y = jnp.maximum(t1 - t2, 0.0)
                sums[c] = sums[c] + y
                sqs[c] = sqs[c] + y * y
                ys.append(y)
            m1 = counts == j0
            olds = [plsc.load_gather(aggs, [dst16 + c]) for c in range(CPW)]
            for c in range(CPW):
                plsc.store_scatter(aggs, [dst16 + c],
                                   jnp.maximum(olds[c], ys[c]), mask=m1)

            @pl.when(dup)
            def _slow_path():
                def cond(j):
                    return jnp.any(counts >= j)

                def pass_body(j):
                    m = counts == j
                    for c in range(CPW):
                        old = plsc.load_gather(aggs, [dst16 + c])
                        plsc.store_scatter(aggs, [dst16 + c],
                                           jnp.maximum(old, ys[c]), mask=m)
                    return j + 1

                lax.while_loop(cond, pass_body, j0 + 1)

            return tuple(sums) + tuple(sqs) + nxt
        return vec_body

    vb0 = vec_body_for(srcb0, dstb0)
    vb1 = vec_body_for(srcb1, dstb1)

    issue(0, srcb0, dstb0, sem0)

    def prologue(sbuf, dbuf):
        s0, d0 = load_idx(sbuf, dbuf, 0)
        s1, d1 = load_idx(sbuf, dbuf, 1)
        return (s0, d0) + count_of(d0) + (s1, d1)

    def pair_body(p, carry):
        issue(2 * p + 1, srcb1, dstb1, sem1)
        wait(srcb0, dstb0, sem0)
        carry = carry[:2 * CPW] + prologue(srcb0, dstb0)
        carry = lax.fori_loop(0, NVEC, vb0, carry, unroll=2)

        @pl.when(p < NPAIR - 1)
        def _prefetch_chunk():
            issue(2 * p + 2, srcb0, dstb0, sem0)

        wait(srcb1, dstb1, sem1)
        carry = carry[:2 * CPW] + prologue(srcb1, dstb1)
        carry = lax.fori_loop(0, NVEC, vb1, carry, unroll=2)
        return carry

    zero = jnp.zeros((LANES,), jnp.float32)
    zi = jnp.zeros((LANES,), jnp.int32)
    carry0 = ((zero,) * (2 * CPW)
              + (zi, zi, zi, jnp.bool_(False), zi, zi))
    carry = lax.fori_loop(0, NPAIR, pair_body, carry0)
    for j in range(2 * CPW):
        statsb[pl.ds(j * LANES, LANES)] = carry[j]
    pltpu.sync_copy(statsb, stats_hbm.at[wid])
    pltpu.sync_copy(aggs, agg_hbm.at[wid])


def _phase2(T1t, T2t, src, dst):
    mesh = plsc.VectorSubcoreMesh(core_axis_name="c", subcore_axis_name="s")
    kfn = pl.kernel(
        _phase2_body,
        out_type=(jax.ShapeDtypeStruct((NW, N * CPW), jnp.float32),
                  jax.ShapeDtypeStruct((NW, 2 * CPW * LANES), jnp.float32)),
        mesh=mesh,
        compiler_params=pltpu.CompilerParams(needs_layout_passes=False),
        scratch_types=[
            pltpu.VMEM((N * CPW,), jnp.float32),
            pltpu.VMEM((N * CPW,), jnp.float32),
            pltpu.VMEM((N * CPW,), jnp.float32),
            pltpu.VMEM((CH,), jnp.int32),
            pltpu.VMEM((CH,), jnp.int32),
            pltpu.VMEM((CH,), jnp.int32),
            pltpu.VMEM((CH,), jnp.int32),
            pltpu.VMEM((2 * CPW * LANES,), jnp.float32),
            pltpu.SemaphoreType.DMA,
            pltpu.SemaphoreType.DMA,
        ],
    )
    return kfn(T1t, T2t, src, dst)


def _phase3_body(agg_ref, stats_ref, x_ref, ge1_ref, bee1_ref, Wu1_ref,
                 bu1_ref, gu1_ref, beu1_ref, Wu2_ref, bu2_ref, out_ref):
    stats = stats_ref[...]
    kk = lax.broadcasted_iota(jnp.int32, (D, NW), 0)
    ww = lax.broadcasted_iota(jnp.int32, (D, NW), 1)
    G1 = jnp.where(kk // CPW == ww, 1.0, 0.0)
    R = jnp.dot(G1, stats, preferred_element_type=jnp.float32, precision=lax.Precision.HIGHEST)
    km = lax.broadcasted_iota(jnp.int32, (D, D), 0)
    mm = lax.broadcasted_iota(jnp.int32, (D, D), 1)
    msk_s = mm // LANES == km % CPW
    msk_q = mm // LANES == CPW + km % CPW
    S_col = jnp.sum(jnp.where(msk_s, R, 0.0), axis=1, keepdims=True)
    Q_col = jnp.sum(jnp.where(msk_q, R, 0.0), axis=1, keepdims=True)
    I_d = jnp.where(km == mm, 1.0, 0.0)
    ones_row = jnp.ones((1, D), jnp.float32)
    S = jnp.dot(ones_row, S_col * I_d, preferred_element_type=jnp.float32, precision=lax.Precision.HIGHEST)
    Q = jnp.dot(ones_row, Q_col * I_d, preferred_element_type=jnp.float32, precision=lax.Precision.HIGHEST)
    em = S / E
    ev = jnp.maximum(Q / E - em * em, 0.0)
    agg = agg_ref[...]
    agg_bn = ge1_ref[...] * (agg - em) * lax.rsqrt(ev + EPS) + bee1_ref[...]
    u = jnp.maximum(_mmul(agg_bn, Wu1_ref[...]) + bu1_ref[...], 0.0)
    um = jnp.mean(u, axis=0, keepdims=True)
    uv = jnp.mean((u - um) ** 2, axis=0, keepdims=True)
    ub = gu1_ref[...] * (u - um) * lax.rsqrt(uv + EPS) + beu1_ref[...]
    out_ref[...] = _mmul(ub, Wu2_ref[...]) + bu2_ref[...] + x_ref[...]


def _phase3(agg, stats, x, g_e1, be_e1, W_u1, b_u1, g_u1, be_u1, W_u2, b_u2):
    return pl.pallas_call(
        _phase3_body,
        out_shape=jax.ShapeDtypeStruct((N, D), jnp.float32),
    )(agg, stats, x, g_e1, be_e1, W_u1, b_u1, g_u1, be_u1, W_u2, b_u2)


def kernel(x, coords, edges, not_used, W_off1, b_off1, g_off1, be_off1,
           W_off2, b_off2, W_e1, b_e1, g_e1, be_e1, W_u1, b_u1, g_u1, be_u1,
           W_u2, b_u2):
    src = edges[:, 0] * CPW
    dst = edges[:, 1] * CPW
    A = W_e1[:D]
    B = W_e1[D:]
    row = lambda v: v.reshape(1, -1)
    T1, T2 = _phase1(x, coords, W_off1, row(b_off1), row(g_off1),
                     row(be_off1), W_off2, row(b_off2), A, B, row(b_e1))
    to_t = lambda T: (T.reshape(N, NW, CPW).transpose(1, 0, 2)
                      .reshape(NW, N * CPW))
    aggt, stats = _phase2(to_t(T1), to_t(T2), src, dst)
    agg = aggt.reshape(NW, N, CPW).transpose(1, 0, 2).reshape(N, D)
    out = _phase3(agg, stats, x, row(g_e1), row(be_e1), W_u1, row(b_u1),
                  row(g_u1), row(be_u1), W_u2, row(b_u2))
    return out


# speedup vs baseline: 4.1499x; 1.0272x over previous
import functools

import jax
import jax.numpy as jnp
from jax import lax
from jax.experimental import pallas as pl
from jax.experimental.pallas import tpu as pltpu
from jax.experimental.pallas import tpu_sc as plsc

N = 10000
E = 320000
D = 128
EPS = 1e-3

NC = 2
NS = 16
NW = NC * NS
CPW = D // NW
CH = 1280
NBKT = N // 2
LANES = 16


def _mmul(a, b):
    return jnp.dot(a.astype(jnp.bfloat16), b.astype(jnp.bfloat16),
                   preferred_element_type=jnp.float32)


def _phase1_body(x_ref, coords_ref, Woff1_ref, boff1_ref, goff1_ref,
                 beoff1_ref, Woff2_ref, boff2_ref, A_ref, B_ref, be1_ref,
                 T1_ref, T2_ref):
    x = x_ref[...]
    coords = coords_ref[...]
    h = jnp.maximum(_mmul(x, Woff1_ref[...]) + boff1_ref[...], 0.0)
    mean = jnp.mean(h, axis=0, keepdims=True)
    var = jnp.mean((h - mean) ** 2, axis=0, keepdims=True)
    hb = goff1_ref[...] * (h - mean) * lax.rsqrt(var + EPS) + beoff1_ref[...]
    offset = _mmul(hb, Woff2_ref[...]) + boff2_ref[...]
    co = coords + offset
    T1_ref[...] = (_mmul(x, A_ref[...]) + _mmul(coords, B_ref[...])
                   + be1_ref[...])
    T2_ref[...] = _mmul(co, B_ref[...])


def _phase1(x, coords, W_off1, b_off1, g_off1, be_off1, W_off2, b_off2,
            A, B, b_e1):
    return pl.pallas_call(
        _phase1_body,
        out_shape=(jax.ShapeDtypeStruct((N, D), jnp.float32),
                   jax.ShapeDtypeStruct((N, D), jnp.float32)),
    )(x, coords, W_off1, b_off1, g_off1, be_off1, W_off2, b_off2, A, B, b_e1)


NCHUNK = E // CH
NPAIR = NCHUNK // 2


def _phase2_body(T1_hbm, T2_hbm, src_hbm, dst_hbm,
                 agg_hbm, stats_hbm, T1s, T2s, aggs,
                 srcb0, dstb0, srcb1, dstb1, idbuf, statsb, sem0, sem1):
    wid = lax.axis_index("s") * NC + lax.axis_index("c")
    pltpu.sync_copy(T1_hbm.at[wid], T1s)
    pltpu.sync_copy(T2_hbm.at[wid], T2s)
    neg = jnp.full((LANES,), -jnp.inf, jnp.float32)

    def init_body(i, _):
        aggs[pl.ds(pl.multiple_of(i * LANES, LANES), LANES)] = neg
        return 0
    lax.fori_loop(0, (N * CPW) // LANES, init_body, 0)

    probe, _ = plsc.scan_count(lax.iota(jnp.int32, LANES))
    j0 = jnp.min(probe)

    def issue(ci, sb, db, sem):
        off = ci * CH
        pltpu.async_copy(src_hbm.at[pl.ds(off, CH)], sb, sem)
        pltpu.async_copy(dst_hbm.at[pl.ds(off, CH)], db, sem)

    def wait(sb, db, sem):
        pltpu.make_async_copy(src_hbm.at[pl.ds(0, CH)], sb, sem).wait()
        pltpu.make_async_copy(dst_hbm.at[pl.ds(0, CH)], db, sem).wait()

    NVEC = CH // LANES

    def load_idx(sbuf, dbuf, i):
        base = pl.multiple_of(i * LANES, LANES)
        return sbuf[pl.ds(base, LANES)], dbuf[pl.ds(base, LANES)]

    lane = lax.iota(jnp.int32, LANES)

    def detect(d16):
        b16 = lax.shift_right_logical(d16, 3)
        plsc.store_scatter(idbuf, [b16], lane)
        got = plsc.load_gather(idbuf, [b16])
        return (jnp.any(got != lane),)

    def vec_body_for(sbuf, dbuf):
        def vec_body(i, carry):
            sums = list(carry[:CPW])
            sqs = list(carry[CPW:2 * CPW])
            src16, dst16, dup, s_n, d_n = carry[2 * CPW:]
            dup_n = detect(d_n)
            sd_nn = load_idx(sbuf, dbuf, jnp.minimum(i + 2, NVEC - 1))
            nxt = (s_n, d_n) + dup_n + sd_nn
            ys = []
            for c in range(CPW):
                t1 = plsc.load_gather(T1s, [src16 + c])
                t2 = plsc.load_gather(T2s, [dst16 + c])
                y = jnp.maximum(t1 - t2, 0.0)
                sums[c] = sums[c] + y
                sqs[c] = sqs[c] + y * y
                ys.append(y)
            m1 = jnp.logical_not(jnp.broadcast_to(dup, (LANES,)))
            olds = [plsc.load_gather(aggs, [dst16 + c]) for c in range(CPW)]
            for c in range(CPW):
                plsc.store_scatter(aggs, [dst16 + c],
                                   jnp.maximum(olds[c], ys[c]), mask=m1)

            @pl.when(dup)
            def _slow_path():
                counts, _ = plsc.scan_count(dst16)

                def cond(j):
                    return jnp.any(counts >= j)

                def pass_body(j):
                    m = counts == j
                    for c in range(CPW):
                        old = plsc.load_gather(aggs, [dst16 + c])
                        plsc.store_scatter(aggs, [dst16 + c],
                                           jnp.maximum(old, ys[c]), mask=m)
                    return j + 1

                lax.while_loop(cond, pass_body, j0)

            return tuple(sums) + tuple(sqs) + nxt
        return vec_body

    vb0 = vec_body_for(srcb0, dstb0)
    vb1 = vec_body_for(srcb1, dstb1)

    issue(0, srcb0, dstb0, sem0)

    def prologue(sbuf, dbuf):
        s0, d0 = load_idx(sbuf, dbuf, 0)
        s1, d1 = load_idx(sbuf, dbuf, 1)
        return (s0, d0) + detect(d0) + (s1, d1)

    def pair_body(p, carry):
        issue(2 * p + 1, srcb1, dstb1, sem1)
        wait(srcb0, dstb0, sem0)
        carry = carry[:2 * CPW] + prologue(srcb0, dstb0)
        carry = lax.fori_loop(0, NVEC, vb0, carry, unroll=2)

        @pl.when(p < NPAIR - 1)
        def _prefetch_chunk():
            issue(2 * p + 2, srcb0, dstb0, sem0)

        wait(srcb1, dstb1, sem1)
        carry = carry[:2 * CPW] + prologue(srcb1, dstb1)
        carry = lax.fori_loop(0, NVEC, vb1, carry, unroll=2)
        return carry

    zero = jnp.zeros((LANES,), jnp.float32)
    zi = jnp.zeros((LANES,), jnp.int32)
    carry0 = ((zero,) * (2 * CPW)
              + (zi, zi, jnp.bool_(False), zi, zi))
    carry = lax.fori_loop(0, NPAIR, pair_body, carry0)
    for j in range(2 * CPW):
        statsb[pl.ds(j * LANES, LANES)] = carry[j]
    pltpu.sync_copy(statsb, stats_hbm.at[wid])
    pltpu.sync_copy(aggs, agg_hbm.at[wid])


def _phase2(T1t, T2t, src, dst):
    mesh = plsc.VectorSubcoreMesh(core_axis_name="c", subcore_axis_name="s")
    kfn = pl.kernel(
        _phase2_body,
        out_type=(jax.ShapeDtypeStruct((NW, N * CPW), jnp.float32),
                  jax.ShapeDtypeStruct((NW, 2 * CPW * LANES), jnp.float32)),
        mesh=mesh,
        compiler_params=pltpu.CompilerParams(needs_layout_passes=False),
        scratch_types=[
            pltpu.VMEM((N * CPW,), jnp.float32),
            pltpu.VMEM((N * CPW,), jnp.float32),
            pltpu.VMEM((N * CPW,), jnp.float32),
            pltpu.VMEM((CH,), jnp.int32),
            pltpu.VMEM((CH,), jnp.int32),
            pltpu.VMEM((CH,), jnp.int32),
            pltpu.VMEM((CH,), jnp.int32),
            pltpu.VMEM((NBKT,), jnp.int32),
            pltpu.VMEM((2 * CPW * LANES,), jnp.float32),
            pltpu.SemaphoreType.DMA,
            pltpu.SemaphoreType.DMA,
        ],
    )
    return kfn(T1t, T2t, src, dst)


def _phase3_body(agg_ref, stats_ref, x_ref, ge1_ref, bee1_ref, Wu1_ref,
                 bu1_ref, gu1_ref, beu1_ref, Wu2_ref, bu2_ref, out_ref):
    stats = stats_ref[...]
    kk = lax.broadcasted_iota(jnp.int32, (D, NW), 0)
    ww = lax.broadcasted_iota(jnp.int32, (D, NW), 1)
    G1 = jnp.where(kk // CPW == ww, 1.0, 0.0)
    R = jnp.dot(G1, stats, preferred_element_type=jnp.float32, precision=lax.Precision.HIGHEST)
    km = lax.broadcasted_iota(jnp.int32, (D, D), 0)
    mm = lax.broadcasted_iota(jnp.int32, (D, D), 1)
    msk_s = mm // LANES == km % CPW
    msk_q = mm // LANES == CPW + km % CPW
    S_col = jnp.sum(jnp.where(msk_s, R, 0.0), axis=1, keepdims=True)
    Q_col = jnp.sum(jnp.where(msk_q, R, 0.0), axis=1, keepdims=True)
    I_d = jnp.where(km == mm, 1.0, 0.0)
    ones_row = jnp.ones((1, D), jnp.float32)
    S = jnp.dot(ones_row, S_col * I_d, preferred_element_type=jnp.float32, precision=lax.Precision.HIGHEST)
    Q = jnp.dot(ones_row, Q_col * I_d, preferred_element_type=jnp.float32, precision=lax.Precision.HIGHEST)
    em = S / E
    ev = jnp.maximum(Q / E - em * em, 0.0)
    agg = agg_ref[...]
    agg_bn = ge1_ref[...] * (agg - em) * lax.rsqrt(ev + EPS) + bee1_ref[...]
    u = jnp.maximum(_mmul(agg_bn, Wu1_ref[...]) + bu1_ref[...], 0.0)
    um = jnp.mean(u, axis=0, keepdims=True)
    uv = jnp.mean((u - um) ** 2, axis=0, keepdims=True)
    ub = gu1_ref[...] * (u - um) * lax.rsqrt(uv + EPS) + beu1_ref[...]
    out_ref[...] = _mmul(ub, Wu2_ref[...]) + bu2_ref[...] + x_ref[...]


def _phase3(agg, stats, x, g_e1, be_e1, W_u1, b_u1, g_u1, be_u1, W_u2, b_u2):
    return pl.pallas_call(
        _phase3_body,
        out_shape=jax.ShapeDtypeStruct((N, D), jnp.float32),
    )(agg, stats, x, g_e1, be_e1, W_u1, b_u1, g_u1, be_u1, W_u2, b_u2)


def kernel(x, coords, edges, not_used, W_off1, b_off1, g_off1, be_off1,
           W_off2, b_off2, W_e1, b_e1, g_e1, be_e1, W_u1, b_u1, g_u1, be_u1,
           W_u2, b_u2):
    src = edges[:, 0] * CPW
    dst = edges[:, 1] * CPW
    A = W_e1[:D]
    B = W_e1[D:]
    row = lambda v: v.reshape(1, -1)
    T1, T2 = _phase1(x, coords, W_off1, row(b_off1), row(g_off1),
                     row(be_off1), W_off2, row(b_off2), A, B, row(b_e1))
    to_t = lambda T: (T.reshape(N, NW, CPW).transpose(1, 0, 2)
                      .reshape(NW, N * CPW))
    aggt, stats = _phase2(to_t(T1), to_t(T2), src, dst)
    agg = aggt.reshape(NW, N, CPW).transpose(1, 0, 2).reshape(N, D)
    out = _phase3(agg, stats, x, row(g_e1), row(be_e1), W_u1, row(b_u1),
                  row(g_u1), row(be_u1), W_u2, row(b_u2))
    return out
